# Initial kernel scaffold; baseline (speedup 1.0000x reference)
import jax
import jax.numpy as jnp
from jax.experimental import pallas as pl


def kernel(relative_embeddings, length):
    raise NotImplementedError("write your pallas kernel here")


# speedup vs baseline: 2.3792x; 2.3792x over previous
import functools

import jax
import jax.numpy as jnp
from jax import lax
from jax.experimental import pallas as pl
from jax.experimental.pallas import tpu as pltpu
from jax.experimental.pallas import tpu_sc as plsc

D_MODEL = 256
MAX_REL = 128
LENGTH = 512
V_ROWS = 2 * MAX_REL + 1
PAD = LENGTH - MAX_REL - 1
B_ROWS = 2 * PAD + V_ROWS
NUM_CORES = 2
NUM_SUBCORES = 16
NUM_WORKERS = NUM_CORES * NUM_SUBCORES
ROWS_PER_WORKER = LENGTH // NUM_WORKERS
PAD_STEPS = -(-PAD // NUM_SUBCORES)


def _sc_body(table_hbm, out_hbm, b_sh, sem):
    s = lax.axis_index("s")
    c = lax.axis_index("c")
    wid = s * NUM_CORES + c

    @pl.when(s == 0)
    def _():
        pltpu.sync_copy(table_hbm, b_sh.at[pl.ds(PAD, V_ROWS)])

    def _pad_body(k, carry):
        p = s + k * NUM_SUBCORES

        @pl.when(p < PAD)
        def _():
            pltpu.sync_copy(table_hbm.at[pl.ds(0, 1)], b_sh.at[pl.ds(p, 1)])
            pltpu.sync_copy(
                table_hbm.at[pl.ds(V_ROWS - 1, 1)],
                b_sh.at[pl.ds(PAD + V_ROWS + p, 1)],
            )

        return carry

    lax.fori_loop(0, PAD_STEPS, _pad_body, 0)
    plsc.subcore_barrier()

    copies = []
    for r in range(ROWS_PER_WORKER):
        i = wid * ROWS_PER_WORKER + r
        copies.append(
            pltpu.async_copy(
                b_sh.at[pl.ds((LENGTH - 1) - i, LENGTH)],
                out_hbm.at[pl.ds(i * LENGTH, LENGTH)],
                sem,
            )
        )
    for cp in copies:
        cp.wait()


@jax.jit
def _rel_pos_gather(table):
    mesh = plsc.VectorSubcoreMesh(
        core_axis_name="c",
        subcore_axis_name="s",
        num_cores=NUM_CORES,
        num_subcores=NUM_SUBCORES,
    )
    run = functools.partial(
        pl.kernel,
        out_type=jax.ShapeDtypeStruct((LENGTH * LENGTH, D_MODEL), jnp.float32),
        mesh=mesh,
        scratch_types=[
            pltpu.VMEM_SHARED((B_ROWS, D_MODEL), jnp.float32),
            pltpu.SemaphoreType.DMA,
        ],
        compiler_params=pltpu.CompilerParams(use_tc_tiling_on_sc=False),
    )(_sc_body)
    return run(table)


def kernel(relative_embeddings, length):
    del length
    out_flat = _rel_pos_gather(relative_embeddings)
    return out_flat.reshape(LENGTH, LENGTH, D_MODEL)


# speedup vs baseline: 2.3917x; 1.0053x over previous
import functools

import jax
import jax.numpy as jnp
from jax import lax
from jax.experimental import pallas as pl
from jax.experimental.pallas import tpu as pltpu
from jax.experimental.pallas import tpu_sc as plsc

D_MODEL = 256
MAX_REL = 128
LENGTH = 512
V_ROWS = 2 * MAX_REL + 1
PAD = LENGTH - MAX_REL - 1
B_ROWS = 2 * PAD + V_ROWS
NUM_CORES = 2
NUM_SUBCORES = 16
NUM_WORKERS = NUM_CORES * NUM_SUBCORES
ROWS_PER_WORKER = LENGTH // NUM_WORKERS
PAD_STEPS = -(-PAD // NUM_SUBCORES)


def _sc_body(table_hbm, out_hbm, b_sh, sem):
    s = lax.axis_index("s")
    c = lax.axis_index("c")
    wid = s * NUM_CORES + c

    @pl.when(s == 0)
    def _():
        pltpu.sync_copy(table_hbm, b_sh.at[pl.ds(PAD, V_ROWS)])

    def _pad_body(k, carry):
        p = s + k * NUM_SUBCORES

        @pl.when(p < PAD)
        def _():
            pltpu.sync_copy(table_hbm.at[pl.ds(0, 1)], b_sh.at[pl.ds(p, 1)])
            pltpu.sync_copy(
                table_hbm.at[pl.ds(V_ROWS - 1, 1)],
                b_sh.at[pl.ds(PAD + V_ROWS + p, 1)],
            )

        return carry

    lax.fori_loop(0, PAD_STEPS, _pad_body, 0)
    plsc.subcore_barrier()

    copies = []
    for r in range(ROWS_PER_WORKER):
        i = wid * ROWS_PER_WORKER + r
        copies.append(
            pltpu.async_copy(
                b_sh.at[pl.ds((LENGTH - 1) - i, LENGTH)],
                out_hbm.at[i],
                sem,
            )
        )
    for cp in copies:
        cp.wait()


@jax.jit
def _rel_pos_gather(table):
    mesh = plsc.VectorSubcoreMesh(
        core_axis_name="c",
        subcore_axis_name="s",
        num_cores=NUM_CORES,
        num_subcores=NUM_SUBCORES,
    )
    run = functools.partial(
        pl.kernel,
        out_type=jax.ShapeDtypeStruct((LENGTH, LENGTH, D_MODEL), jnp.float32),
        mesh=mesh,
        scratch_types=[
            pltpu.VMEM_SHARED((B_ROWS, D_MODEL), jnp.float32),
            pltpu.SemaphoreType.DMA,
        ],
        compiler_params=pltpu.CompilerParams(use_tc_tiling_on_sc=False),
    )(_sc_body)
    return run(table)


def kernel(relative_embeddings, length):
    del length
    return _rel_pos_gather(relative_embeddings)


# speedup vs baseline: 6.2017x; 2.5930x over previous
import functools

import jax
import jax.numpy as jnp
from jax import lax
from jax.experimental import pallas as pl
from jax.experimental.pallas import tpu as pltpu
from jax.experimental.pallas import tpu_sc as plsc

D_MODEL = 256
MAX_REL = 128
LENGTH = 512
V_ROWS = 2 * MAX_REL + 1
PAD = LENGTH - MAX_REL - 1
C_ROWS = 1016
NUM_CORES = 2
NUM_SUBCORES = 16
NUM_PASSES = 2
COPIES_PER_PASS = 2
TILES_PER_COPY = NUM_SUBCORES // COPIES_PER_PASS
BUILD_CHUNK = 8
N_CHUNKS = C_ROWS // BUILD_CHUNK
CHUNK_STEPS = -(-N_CHUNKS // TILES_PER_COPY)
M_PER_TILE = 64 // TILES_PER_COPY


def _sc_body(table_hbm, out_hbm, c_sh, tbl_v, stage_v, sem):
    s = lax.axis_index("s")
    c = lax.axis_index("c")
    u = lax.rem(s, COPIES_PER_PASS)
    g = lax.div(s, COPIES_PER_PASS)

    pltpu.sync_copy(table_hbm, tbl_v)

    for p in range(NUM_PASSES):
        r = COPIES_PER_PASS * (NUM_PASSES * c + p) + u

        def _chunk_body(step, carry, r=r):
            ch = g + step * TILES_PER_COPY

            @pl.when(ch < N_CHUNKS)
            def _():
                for j in range(BUILD_CHUNK):
                    k = ch * BUILD_CHUNK + j
                    t = lax.clamp(0, k + 7 - r - PAD, 2 * MAX_REL)
                    for h in range(D_MODEL // 16):
                        stage_v[j, pl.ds(h * 16, 16)] = (
                            tbl_v[t, pl.ds(h * 16, 16)]
                        )
                dst_k = pl.multiple_of(ch * BUILD_CHUNK, BUILD_CHUNK)
                pltpu.sync_copy(stage_v, c_sh.at[u, pl.ds(dst_k, BUILD_CHUNK)])

            return carry

        lax.fori_loop(0, CHUNK_STEPS, _chunk_body, 0)
        plsc.subcore_barrier()

        copies = []
        for jj in range(M_PER_TILE):
            m = g * M_PER_TILE + jj
            i = r + 8 * m
            start = pl.multiple_of(504 - 8 * m, 8)
            copies.append(
                pltpu.async_copy(
                    c_sh.at[u, pl.ds(start, LENGTH)],
                    out_hbm.at[i],
                    sem,
                )
            )
        for cp in copies:
            cp.wait()
        if p + 1 < NUM_PASSES:
            plsc.subcore_barrier()


@jax.jit
def _rel_pos_gather(table):
    mesh = plsc.VectorSubcoreMesh(
        core_axis_name="c",
        subcore_axis_name="s",
        num_cores=NUM_CORES,
        num_subcores=NUM_SUBCORES,
    )
    run = functools.partial(
        pl.kernel,
        out_type=jax.ShapeDtypeStruct((LENGTH, LENGTH, D_MODEL), jnp.float32),
        mesh=mesh,
        scratch_types=[
            pltpu.VMEM_SHARED(
                (COPIES_PER_PASS, C_ROWS, D_MODEL), jnp.float32
            ),
            pltpu.VMEM((V_ROWS, D_MODEL), jnp.float32),
            pltpu.VMEM((BUILD_CHUNK, D_MODEL), jnp.float32),

            pltpu.SemaphoreType.DMA,
        ],
        compiler_params=pltpu.CompilerParams(use_tc_tiling_on_sc=True),
    )(_sc_body)
    return run(table)


def kernel(relative_embeddings, length):
    del length
    return _rel_pos_gather(relative_embeddings)


# speedup vs baseline: 6.6210x; 1.0676x over previous
import functools

import jax
import jax.numpy as jnp
from jax import lax
from jax.experimental import pallas as pl
from jax.experimental.pallas import tpu as pltpu
from jax.experimental.pallas import tpu_sc as plsc

D_MODEL = 256
MAX_REL = 128
LENGTH = 512
V_ROWS = 2 * MAX_REL + 1
PAD = LENGTH - MAX_REL - 1
C_ROWS = 1016
NUM_CORES = 2
NUM_SUBCORES = 16
NUM_COPIES = 4
BUILD_ROWS = 64
LAST_ROWS = C_ROWS - 15 * BUILD_ROWS
M_PER_TILE = 4


def _build_copy(r, s, buf_ref, tbl_v, stage_v):
    n_rows = jnp.where(s == NUM_SUBCORES - 1, LAST_ROWS, BUILD_ROWS)

    def _row_body(j, carry):
        k = s * BUILD_ROWS + j
        t = lax.clamp(0, k + 7 - r - PAD, 2 * MAX_REL)
        for h in range(D_MODEL // 16):
            stage_v[j, pl.ds(h * 16, 16)] = tbl_v[t, pl.ds(h * 16, 16)]
        return carry

    lax.fori_loop(0, n_rows, _row_body, 0)
    dst_k = pl.multiple_of(s * BUILD_ROWS, 8)

    @pl.when(s < NUM_SUBCORES - 1)
    def _():
        pltpu.sync_copy(stage_v, buf_ref.at[pl.ds(dst_k, BUILD_ROWS)])

    @pl.when(s == NUM_SUBCORES - 1)
    def _():
        pltpu.sync_copy(
            stage_v.at[pl.ds(0, LAST_ROWS)],
            buf_ref.at[pl.ds(dst_k, LAST_ROWS)],
        )


def _fire_streams(r, s, buf_ref, out_hbm, sem):
    copies = []
    for jj in range(M_PER_TILE):
        m = s * M_PER_TILE + jj
        i = r + 8 * m
        start = pl.multiple_of(504 - 8 * m, 8)
        copies.append(
            pltpu.async_copy(
                buf_ref.at[pl.ds(start, LENGTH)], out_hbm.at[i], sem
            )
        )
    return copies


def _sc_body(table_hbm, out_hbm, c_sh, tbl_v, stage_v, sem_a, sem_b):
    s = lax.axis_index("s")
    c = lax.axis_index("c")
    buf0, buf1 = c_sh.at[0], c_sh.at[1]

    pltpu.sync_copy(table_hbm, tbl_v)

    def res(q):
        return NUM_COPIES * c + q

    _build_copy(res(0), s, buf0, tbl_v, stage_v)
    _build_copy(res(1), s, buf1, tbl_v, stage_v)
    plsc.subcore_barrier()

    st0 = _fire_streams(res(0), s, buf0, out_hbm, sem_a)
    st1 = _fire_streams(res(1), s, buf1, out_hbm, sem_b)
    for cp in st0:
        cp.wait()
    plsc.subcore_barrier()

    _build_copy(res(2), s, buf0, tbl_v, stage_v)
    plsc.subcore_barrier()

    st2 = _fire_streams(res(2), s, buf0, out_hbm, sem_a)
    for cp in st1:
        cp.wait()
    plsc.subcore_barrier()

    _build_copy(res(3), s, buf1, tbl_v, stage_v)
    plsc.subcore_barrier()

    st3 = _fire_streams(res(3), s, buf1, out_hbm, sem_b)
    for cp in st2:
        cp.wait()
    for cp in st3:
        cp.wait()


@jax.jit
def _rel_pos_gather(table):
    mesh = plsc.VectorSubcoreMesh(
        core_axis_name="c",
        subcore_axis_name="s",
        num_cores=NUM_CORES,
        num_subcores=NUM_SUBCORES,
    )
    run = functools.partial(
        pl.kernel,
        out_type=jax.ShapeDtypeStruct((LENGTH, LENGTH, D_MODEL), jnp.float32),
        mesh=mesh,
        scratch_types=[
            pltpu.VMEM_SHARED((2, C_ROWS, D_MODEL), jnp.float32),
            pltpu.VMEM((V_ROWS, D_MODEL), jnp.float32),
            pltpu.VMEM((BUILD_ROWS, D_MODEL), jnp.float32),
            pltpu.SemaphoreType.DMA,
            pltpu.SemaphoreType.DMA,
        ],
        compiler_params=pltpu.CompilerParams(
            use_tc_tiling_on_sc=True,
            internal_scratch_in_bytes=128 * 1024,
        ),
    )(_sc_body)
    return run(table)


def kernel(relative_embeddings, length):
    del length
    return _rel_pos_gather(relative_embeddings)


# speedup vs baseline: 9.6385x; 1.4557x over previous
import functools

import jax
import jax.numpy as jnp
from jax import lax
from jax.experimental import pallas as pl
from jax.experimental.pallas import tpu as pltpu
from jax.experimental.pallas import tpu_sc as plsc

D_MODEL = 256
MAX_REL = 128
LENGTH = 512
V_ROWS = 2 * MAX_REL + 1
V_PAD = 264
PAD = LENGTH - MAX_REL - 1
C_ROWS = 1016
STAGE_ROW = 384
HALF = 128
NUM_CORES = 2
NUM_SUBCORES = 16
M_PER_TILE = 32
LANES = 16


def _sc_body(table_hbm, out_hbm, buf_v, sem):
    s = lax.axis_index("s")
    c = lax.axis_index("c")
    w = s * NUM_CORES + c
    r = lax.div(w, 4)
    rest = lax.rem(w, 4)
    d0 = lax.rem(rest, 2) * HALF
    m0 = lax.div(rest, 2) * M_PER_TILE

    pltpu.sync_copy(
        table_hbm.at[pl.ds(0, V_PAD), pl.ds(pl.multiple_of(d0, HALF), HALF)],
        buf_v.at[pl.ds(STAGE_ROW, V_PAD)],
    )

    lo = PAD - 7 + r
    hi = lo + V_ROWS

    def _move(k, carry):
        src = k + 8 - r
        for h in range(HALF // LANES):
            buf_v[k, pl.ds(h * LANES, LANES)] = (
                buf_v[src, pl.ds(h * LANES, LANES)]
            )
        return carry

    lax.fori_loop(lo, hi, _move, 0)

    bot = [buf_v[lo, pl.ds(h * LANES, LANES)] for h in range(HALF // LANES)]
    top = [
        buf_v[hi - 1, pl.ds(h * LANES, LANES)] for h in range(HALF // LANES)
    ]

    def _fill_bot(k, carry):
        for h in range(HALF // LANES):
            buf_v[k, pl.ds(h * LANES, LANES)] = bot[h]
        return carry

    def _fill_top(k, carry):
        for h in range(HALF // LANES):
            buf_v[k, pl.ds(h * LANES, LANES)] = top[h]
        return carry

    lax.fori_loop(0, lo, _fill_bot, 0)
    lax.fori_loop(hi, C_ROWS, _fill_top, 0)

    copies = []
    for jj in range(M_PER_TILE):
        m = m0 + jj
        i = r + 8 * m
        start = pl.multiple_of(504 - 8 * m, 8)
        copies.append(
            pltpu.async_copy(
                buf_v.at[pl.ds(start, LENGTH)],
                out_hbm.at[
                    i,
                    pl.ds(0, LENGTH),
                    pl.ds(pl.multiple_of(d0, HALF), HALF),
                ],
                sem,
            )
        )
    for cp in copies:
        cp.wait()


@jax.jit
def _rel_pos_gather(table):
    mesh = plsc.VectorSubcoreMesh(
        core_axis_name="c",
        subcore_axis_name="s",
        num_cores=NUM_CORES,
        num_subcores=NUM_SUBCORES,
    )
    run = functools.partial(
        pl.kernel,
        out_type=jax.ShapeDtypeStruct((LENGTH, LENGTH, D_MODEL), jnp.float32),
        mesh=mesh,
        scratch_types=[
            pltpu.VMEM((C_ROWS, HALF), jnp.float32),
            pltpu.SemaphoreType.DMA,
        ],
        compiler_params=pltpu.CompilerParams(use_tc_tiling_on_sc=True),
    )(_sc_body)
    padded = jnp.pad(table, ((0, V_PAD - V_ROWS), (0, 0)))
    return run(padded)


def kernel(relative_embeddings, length):
    del length
    return _rel_pos_gather(relative_embeddings)
